# baseline (speedup 1.0000x reference)
import functools

import jax
import jax.numpy as jnp
from jax import lax
from jax.experimental import pallas as pl
from jax.experimental.pallas import tpu as pltpu
from jax.experimental.pallas import tpu_sc as plsc

N = 10000
E = 320000
D1 = 128
DH = 64
DO = 40
DOP = 48

NC, NS, L = 2, 16, 16
NW = NC * NS
EW = E // NW
K = 125
NCH = EW // K
NBUF = 5
SLA = 2
NP = 10240
RPT = NP // NS

R = 2000
GRID = N // R

_f32 = jnp.float32
_SC_PARAMS = pltpu.CompilerParams(use_tc_tiling_on_sc=False)


@functools.cache
def _make_deg():
    mesh = plsc.VectorSubcoreMesh(core_axis_name="c", subcore_axis_name="s")

    @functools.partial(
        pl.kernel,
        out_type=jax.ShapeDtypeStruct((NC, NP), _f32),
        mesh=mesh,
        compiler_params=_SC_PARAMS,
        scratch_types=[
            pltpu.VMEM((NCH, K), jnp.int32),
            pltpu.VMEM((128,), _f32),
            pltpu.VMEM((NP,), _f32),
            pltpu.VMEM_SHARED((NP,), _f32),
        ],
    )
    def deg_kernel(ei_hbm, out_hbm, idx_v, ones_v, zbuf_v, acc_sh):
        c = lax.axis_index("c")
        s = lax.axis_index("s")
        w = c * NS + s

        @pl.loop(0, 128 // L)
        def _(i):
            ones_v[pl.ds(i * L, L)] = jnp.ones((L,), _f32)

        @pl.when(s == 0)
        def _():
            @pl.loop(0, NP // L)
            def _(i):
                zbuf_v[pl.ds(i * L, L)] = jnp.zeros((L,), _f32)

            pltpu.sync_copy(zbuf_v, acc_sh)

        pltpu.sync_copy(ei_hbm.at[1, w], idx_v)
        plsc.subcore_barrier()

        @pl.loop(0, NCH)
        def _(j):
            pltpu.sync_copy(ones_v.at[pl.ds(0, K)], acc_sh.at[idx_v.at[j]],
                            add=True)

        plsc.subcore_barrier()

        @pl.when(s == 0)
        def _():
            pltpu.sync_copy(acc_sh, out_hbm.at[c])

    return deg_kernel


@functools.cache
def _make_agg(D, nphase):
    zrows = 128
    mesh = plsc.VectorSubcoreMesh(core_axis_name="c", subcore_axis_name="s")

    @functools.partial(
        pl.kernel,
        out_type=jax.ShapeDtypeStruct((nphase, NC, NP, D), _f32),
        mesh=mesh,
        compiler_params=_SC_PARAMS,
        scratch_types=[
            pltpu.VMEM((NCH, K), jnp.int32),
            pltpu.VMEM((NCH, K), jnp.int32),
            [pltpu.VMEM((K, D), _f32)] * NBUF,
            pltpu.VMEM((zrows, D), _f32),
            pltpu.VMEM_SHARED((NP, D), _f32),
            [pltpu.SemaphoreType.DMA] * NBUF,
            pltpu.SemaphoreType.DMA,
        ],
    )
    def agg(*refs):
        g_hbms = refs[:nphase]
        ei_hbm, out_hbm = refs[nphase:nphase + 2]
        sidx, didx, bufs, zbuf, acc_sh, gsems, isem = refs[nphase + 2:]
        c = lax.axis_index("c")
        s = lax.axis_index("s")
        w = c * NS + s

        pltpu.async_copy(ei_hbm.at[0, w], sidx, isem)
        pltpu.async_copy(ei_hbm.at[1, w], didx, isem)

        @pl.loop(0, zrows)
        def _(r):
            for cc in range(D // L):
                zbuf[r, pl.ds(cc * L, L)] = jnp.zeros((L,), _f32)

        pltpu.make_async_copy(ei_hbm.at[0, w], sidx, isem).wait()
        pltpu.make_async_copy(ei_hbm.at[1, w], didx, isem).wait()

        for p in range(nphase):
            g_hbm = g_hbms[p]

            for t in range(RPT // zrows):
                pltpu.sync_copy(
                    zbuf, acc_sh.at[pl.ds(s * RPT + t * zrows, zrows), :])
            plsc.subcore_barrier()

            for b in range(NBUF):
                pltpu.async_copy(g_hbm.at[sidx.at[b]], bufs[b], gsems[b])

            @pl.loop(0, NCH, step=NBUF)
            def _(jj):
                for b in range(NBUF):
                    j = jj + b
                    pltpu.make_async_copy(
                        g_hbm.at[sidx.at[j]], bufs[b], gsems[b]).wait()
                    pltpu.sync_copy(bufs[b], acc_sh.at[didx.at[j]], add=True)

                    @pl.when(j + NBUF < NCH)
                    def _():
                        pltpu.async_copy(g_hbm.at[sidx.at[j + NBUF]],
                                         bufs[b], gsems[b])

            plsc.subcore_barrier()
            pltpu.sync_copy(acc_sh.at[pl.ds(s * RPT, RPT), :],
                            out_hbm.at[p, c, pl.ds(s * RPT, RPT), :])
            if p + 1 < nphase:
                plsc.subcore_barrier()

    return agg


def _dinv_of(degT_ref):
    dT = degT_ref[...]
    d = dT[:, 0:1] + dT[:, 1:2] + 1.0
    return lax.rsqrt(d)


def _g1_body(x_ref, w1_ref, degp_ref, oa_ref, ob_ref):
    dinv = _dinv_of(degp_ref)
    g = jnp.dot(x_ref[...].astype(jnp.bfloat16),
                w1_ref[...].astype(jnp.bfloat16),
                preferred_element_type=_f32) * dinv
    oa_ref[...] = g[:, :DH]
    ob_ref[...] = g[:, DH:]


_g1_call = pl.pallas_call(
    _g1_body,
    grid=(GRID,),
    in_specs=[
        pl.BlockSpec((R, D1), lambda i: (i, 0)),
        pl.BlockSpec((D1, D1), lambda i: (0, 0)),
        pl.BlockSpec((R, NC), lambda i: (i, 0)),
    ],
    out_specs=[
        pl.BlockSpec((R, DH), lambda i: (i, 0)),
        pl.BlockSpec((R, DH), lambda i: (i, 0)),
    ],
    out_shape=[
        jax.ShapeDtypeStruct((N, DH), _f32),
        jax.ShapeDtypeStruct((N, DH), _f32),
    ],
)


def _mid_body(s1al_ref, s1bl_ref, s1ar_ref, s1br_ref, g1l_ref, g1r_ref,
              degp_ref, w2_ref, b1_ref, o_ref):
    dinv = _dinv_of(degp_ref)
    zl = s1al_ref[0, 0] + s1bl_ref[0, 0] + g1l_ref[...]
    zr = s1ar_ref[0, 0] + s1br_ref[0, 0] + g1r_ref[...]
    z = dinv * jnp.concatenate([zl, zr], axis=1) + b1_ref[...]
    z = jnp.maximum(z, 0.0)
    o_ref[...] = jnp.dot(z.astype(jnp.bfloat16),
                         w2_ref[...].astype(jnp.bfloat16),
                         preferred_element_type=_f32) * dinv


_mid_call = pl.pallas_call(
    _mid_body,
    grid=(GRID,),
    in_specs=[
        pl.BlockSpec((1, 1, R, DH), lambda i: (0, 0, i, 0)),
        pl.BlockSpec((1, 1, R, DH), lambda i: (0, 1, i, 0)),
        pl.BlockSpec((1, 1, R, DH), lambda i: (1, 0, i, 0)),
        pl.BlockSpec((1, 1, R, DH), lambda i: (1, 1, i, 0)),
        pl.BlockSpec((R, DH), lambda i: (i, 0)),
        pl.BlockSpec((R, DH), lambda i: (i, 0)),
        pl.BlockSpec((R, NC), lambda i: (i, 0)),
        pl.BlockSpec((D1, DOP), lambda i: (0, 0)),
        pl.BlockSpec((1, D1), lambda i: (0, 0)),
    ],
    out_specs=pl.BlockSpec((R, DOP), lambda i: (i, 0)),
    out_shape=jax.ShapeDtypeStruct((N, DOP), _f32),
)


def _final_body(s2a_ref, s2b_ref, g2_ref, degp_ref, b2_ref, o_ref):
    dinv = _dinv_of(degp_ref)
    u = dinv * (s2a_ref[0, 0] + s2b_ref[0, 0] + g2_ref[...]) + b2_ref[...]
    col = lax.broadcasted_iota(jnp.int32, (R, DOP), 1)
    valid = col < DO
    um = jnp.where(valid, u, -jnp.inf)
    m = jnp.max(um, axis=1, keepdims=True)
    ex = jnp.where(valid, jnp.exp(u - m), 0.0)
    lse = jnp.log(jnp.sum(ex, axis=1, keepdims=True))
    o_ref[...] = (u - m - lse)[:, :DO]


_final_call = pl.pallas_call(
    _final_body,
    grid=(GRID,),
    in_specs=[
        pl.BlockSpec((1, 1, R, DOP), lambda i: (0, 0, i, 0)),
        pl.BlockSpec((1, 1, R, DOP), lambda i: (0, 1, i, 0)),
        pl.BlockSpec((R, DOP), lambda i: (i, 0)),
        pl.BlockSpec((R, NC), lambda i: (i, 0)),
        pl.BlockSpec((1, DOP), lambda i: (0, 0)),
    ],
    out_specs=pl.BlockSpec((R, DO), lambda i: (i, 0)),
    out_shape=jax.ShapeDtypeStruct((N, DO), _f32),
)


def kernel(x, edge_index, W1, b1, W2, b2):
    eir = edge_index.astype(jnp.int32).reshape(2, NW, NCH, K)

    degT = jnp.transpose(_make_deg()(eir))

    g1l, g1r = _g1_call(x, W1, degT)
    s1 = _make_agg(DH, 2)(g1l, g1r, eir)

    W2p = jnp.pad(W2, ((0, 0), (0, DOP - DO)))
    b2p = jnp.pad(b2, (0, DOP - DO))
    g2 = _mid_call(s1, s1, s1, s1, g1l, g1r, degT, W2p,
                   b1.reshape(1, D1))
    s2 = _make_agg(DOP, 1)(g2, eir)

    return _final_call(s2, s2, g2, degT, b2p.reshape(1, DOP))


# speedup vs baseline: 1.0699x; 1.0699x over previous
import functools

import jax
import jax.numpy as jnp
from jax import lax
from jax.experimental import pallas as pl
from jax.experimental.pallas import tpu as pltpu
from jax.experimental.pallas import tpu_sc as plsc

N = 10000
E = 320000
D1 = 128
DH = 64
DO = 40
DOP = 48

NC, NS, L = 2, 16, 16
NW = NC * NS
K = 125
ECH = E // K
NCH1 = ECH // NS
NCH2 = ECH // NW
NBUF = 5
NP = 10240
RPT = NP // NS
ZR = 128

R = 2000
GRID = N // R

_f32 = jnp.float32
_SC_PARAMS = pltpu.CompilerParams(use_tc_tiling_on_sc=False)


def _zero_fill(zbuf, D):
    @pl.loop(0, ZR)
    def _(r):
        for cc in range(D // L):
            zbuf[r, pl.ds(cc * L, L)] = jnp.zeros((L,), _f32)


def _zero_acc_stripe(zbuf, acc_sh, s):
    for t in range(RPT // ZR):
        pltpu.sync_copy(zbuf, acc_sh.at[pl.ds(s * RPT + t * ZR, ZR), :])


def _ring_agg(g_hbm, sidx, didx, bufs, gsems, acc_sh, nch):
    for b in range(NBUF):
        pltpu.async_copy(g_hbm.at[sidx.at[b]], bufs[b], gsems[b])

    @pl.loop(0, nch, step=NBUF)
    def _(jj):
        for b in range(NBUF):
            j = jj + b
            pltpu.make_async_copy(
                g_hbm.at[sidx.at[j]], bufs[b], gsems[b]).wait()
            pltpu.sync_copy(bufs[b], acc_sh.at[didx.at[j]], add=True)

            @pl.when(j + NBUF < nch)
            def _():
                pltpu.async_copy(g_hbm.at[sidx.at[j + NBUF]],
                                 bufs[b], gsems[b])


@functools.cache
def _make_deg():
    mesh = plsc.VectorSubcoreMesh(core_axis_name="c", subcore_axis_name="s")

    @functools.partial(
        pl.kernel,
        out_type=jax.ShapeDtypeStruct((NC, NP), _f32),
        mesh=mesh,
        compiler_params=_SC_PARAMS,
        scratch_types=[
            pltpu.VMEM((NCH2, K), jnp.int32),
            pltpu.VMEM((128,), _f32),
            pltpu.VMEM((NP,), _f32),
            pltpu.VMEM_SHARED((NP,), _f32),
        ],
    )
    def deg_kernel(ei_hbm, out_hbm, idx_v, ones_v, zbuf_v, acc_sh):
        c = lax.axis_index("c")
        s = lax.axis_index("s")
        w = c * NS + s

        @pl.loop(0, 128 // L)
        def _(i):
            ones_v[pl.ds(i * L, L)] = jnp.ones((L,), _f32)

        @pl.when(s == 0)
        def _():
            @pl.loop(0, NP // L)
            def _(i):
                zbuf_v[pl.ds(i * L, L)] = jnp.zeros((L,), _f32)

            pltpu.sync_copy(zbuf_v, acc_sh)

        pltpu.sync_copy(ei_hbm.at[1, pl.ds(w * NCH2, NCH2), :], idx_v)
        plsc.subcore_barrier()

        @pl.loop(0, NCH2)
        def _(j):
            pltpu.sync_copy(ones_v.at[pl.ds(0, K)], acc_sh.at[idx_v.at[j]],
                            add=True)

        plsc.subcore_barrier()

        @pl.when(s == 0)
        def _():
            pltpu.sync_copy(acc_sh, out_hbm.at[c])

    return deg_kernel


@functools.cache
def _make_agg1():
    mesh = plsc.VectorSubcoreMesh(core_axis_name="c", subcore_axis_name="s")

    @functools.partial(
        pl.kernel,
        out_type=jax.ShapeDtypeStruct((NC, NP, DH), _f32),
        mesh=mesh,
        compiler_params=_SC_PARAMS,
        scratch_types=[
            pltpu.VMEM((NCH1, K), jnp.int32),
            pltpu.VMEM((NCH1, K), jnp.int32),
            [pltpu.VMEM((K, DH), _f32)] * NBUF,
            pltpu.VMEM((ZR, DH), _f32),
            pltpu.VMEM_SHARED((NP, DH), _f32),
            [pltpu.SemaphoreType.DMA] * NBUF,
            pltpu.SemaphoreType.DMA,
        ],
    )
    def agg1(gl_hbm, gr_hbm, ei_hbm, out_hbm,
             sidx, didx, bufs, zbuf, acc_sh, gsems, isem):
        c = lax.axis_index("c")
        s = lax.axis_index("s")

        pltpu.async_copy(ei_hbm.at[0, pl.ds(s * NCH1, NCH1), :], sidx, isem)
        pltpu.async_copy(ei_hbm.at[1, pl.ds(s * NCH1, NCH1), :], didx, isem)
        _zero_fill(zbuf, DH)
        pltpu.make_async_copy(ei_hbm.at[0, pl.ds(0, NCH1), :],
                              sidx, isem).wait()
        pltpu.make_async_copy(ei_hbm.at[1, pl.ds(0, NCH1), :],
                              didx, isem).wait()

        _zero_acc_stripe(zbuf, acc_sh, s)
        plsc.subcore_barrier()

        @pl.when(c == 0)
        def _():
            _ring_agg(gl_hbm, sidx, didx, bufs, gsems, acc_sh, NCH1)

        @pl.when(c == 1)
        def _():
            _ring_agg(gr_hbm, sidx, didx, bufs, gsems, acc_sh, NCH1)

        plsc.subcore_barrier()
        pltpu.sync_copy(acc_sh.at[pl.ds(s * RPT, RPT), :],
                        out_hbm.at[c, pl.ds(s * RPT, RPT), :])

    return agg1


@functools.cache
def _make_agg2():
    mesh = plsc.VectorSubcoreMesh(core_axis_name="c", subcore_axis_name="s")

    @functools.partial(
        pl.kernel,
        out_type=jax.ShapeDtypeStruct((NC, NP, DOP), _f32),
        mesh=mesh,
        compiler_params=_SC_PARAMS,
        scratch_types=[
            pltpu.VMEM((NCH2, K), jnp.int32),
            pltpu.VMEM((NCH2, K), jnp.int32),
            [pltpu.VMEM((K, DOP), _f32)] * NBUF,
            pltpu.VMEM((ZR, DOP), _f32),
            pltpu.VMEM_SHARED((NP, DOP), _f32),
            [pltpu.SemaphoreType.DMA] * NBUF,
            pltpu.SemaphoreType.DMA,
        ],
    )
    def agg2(g_hbm, ei_hbm, out_hbm,
             sidx, didx, bufs, zbuf, acc_sh, gsems, isem):
        c = lax.axis_index("c")
        s = lax.axis_index("s")
        w = c * NS + s

        pltpu.async_copy(ei_hbm.at[0, pl.ds(w * NCH2, NCH2), :], sidx, isem)
        pltpu.async_copy(ei_hbm.at[1, pl.ds(w * NCH2, NCH2), :], didx, isem)
        _zero_fill(zbuf, DOP)
        pltpu.make_async_copy(ei_hbm.at[0, pl.ds(0, NCH2), :],
                              sidx, isem).wait()
        pltpu.make_async_copy(ei_hbm.at[1, pl.ds(0, NCH2), :],
                              didx, isem).wait()

        _zero_acc_stripe(zbuf, acc_sh, s)
        plsc.subcore_barrier()

        _ring_agg(g_hbm, sidx, didx, bufs, gsems, acc_sh, NCH2)

        plsc.subcore_barrier()
        pltpu.sync_copy(acc_sh.at[pl.ds(s * RPT, RPT), :],
                        out_hbm.at[c, pl.ds(s * RPT, RPT), :])

    return agg2


def _dinv_of(degT_ref):
    dT = degT_ref[...]
    d = dT[:, 0:1] + dT[:, 1:2] + 1.0
    return lax.rsqrt(d)


def _g1_body(x_ref, w1_ref, degT_ref, oa_ref, ob_ref):
    dinv = _dinv_of(degT_ref)
    g = jnp.dot(x_ref[...], w1_ref[...], preferred_element_type=_f32) * dinv
    oa_ref[...] = g[:, :DH]
    ob_ref[...] = g[:, DH:]


_g1_call = pl.pallas_call(
    _g1_body,
    grid=(GRID,),
    in_specs=[
        pl.BlockSpec((R, D1), lambda i: (i, 0)),
        pl.BlockSpec((D1, D1), lambda i: (0, 0)),
        pl.BlockSpec((R, NC), lambda i: (i, 0)),
    ],
    out_specs=[
        pl.BlockSpec((R, DH), lambda i: (i, 0)),
        pl.BlockSpec((R, DH), lambda i: (i, 0)),
    ],
    out_shape=[
        jax.ShapeDtypeStruct((N, DH), _f32),
        jax.ShapeDtypeStruct((N, DH), _f32),
    ],
)


def _mid_body(s1l_ref, s1r_ref, g1l_ref, g1r_ref, degT_ref, w2_ref, b1_ref,
              o_ref):
    dinv = _dinv_of(degT_ref)
    zl = s1l_ref[0] + g1l_ref[...]
    zr = s1r_ref[0] + g1r_ref[...]
    z = dinv * jnp.concatenate([zl, zr], axis=1) + b1_ref[...]
    z = jnp.maximum(z, 0.0)
    o_ref[...] = jnp.dot(z, w2_ref[...], preferred_element_type=_f32) * dinv


_mid_call = pl.pallas_call(
    _mid_body,
    grid=(GRID,),
    in_specs=[
        pl.BlockSpec((1, R, DH), lambda i: (0, i, 0)),
        pl.BlockSpec((1, R, DH), lambda i: (1, i, 0)),
        pl.BlockSpec((R, DH), lambda i: (i, 0)),
        pl.BlockSpec((R, DH), lambda i: (i, 0)),
        pl.BlockSpec((R, NC), lambda i: (i, 0)),
        pl.BlockSpec((D1, DOP), lambda i: (0, 0)),
        pl.BlockSpec((1, D1), lambda i: (0, 0)),
    ],
    out_specs=pl.BlockSpec((R, DOP), lambda i: (i, 0)),
    out_shape=jax.ShapeDtypeStruct((N, DOP), _f32),
)


def _final_body(s2a_ref, s2b_ref, g2_ref, degT_ref, b2_ref, o_ref):
    dinv = _dinv_of(degT_ref)
    u = dinv * (s2a_ref[0] + s2b_ref[0] + g2_ref[...]) + b2_ref[...]
    col = lax.broadcasted_iota(jnp.int32, (R, DOP), 1)
    valid = col < DO
    um = jnp.where(valid, u, -jnp.inf)
    m = jnp.max(um, axis=1, keepdims=True)
    ex = jnp.where(valid, jnp.exp(u - m), 0.0)
    lse = jnp.log(jnp.sum(ex, axis=1, keepdims=True))
    o_ref[...] = (u - m - lse)[:, :DO]


_final_call = pl.pallas_call(
    _final_body,
    grid=(GRID,),
    in_specs=[
        pl.BlockSpec((1, R, DOP), lambda i: (0, i, 0)),
        pl.BlockSpec((1, R, DOP), lambda i: (1, i, 0)),
        pl.BlockSpec((R, DOP), lambda i: (i, 0)),
        pl.BlockSpec((R, NC), lambda i: (i, 0)),
        pl.BlockSpec((1, DOP), lambda i: (0, 0)),
    ],
    out_specs=pl.BlockSpec((R, DO), lambda i: (i, 0)),
    out_shape=jax.ShapeDtypeStruct((N, DO), _f32),
)


def kernel(x, edge_index, W1, b1, W2, b2):
    eir = edge_index.astype(jnp.int32).reshape(2, ECH, K)

    degp = _make_deg()(eir)
    degT = jnp.transpose(degp)

    g1l, g1r = _g1_call(x, W1, degT)
    s1 = _make_agg1()(g1l, g1r, eir)

    W2p = jnp.pad(W2, ((0, 0), (0, DOP - DO)))
    b2p = jnp.pad(b2, (0, DOP - DO))
    g2 = _mid_call(s1, s1, g1l, g1r, degT, W2p, b1.reshape(1, D1))
    s2 = _make_agg2()(g2, eir)

    return _final_call(s2, s2, g2, degT, b2p.reshape(1, DOP))
